# baseline (speedup 1.0000x reference)
import functools

import jax
import jax.numpy as jnp
from jax import lax
from jax.experimental import pallas as pl
from jax.experimental.pallas import tpu as pltpu

B, E, C, HW, K = 32, 64, 768, 576, 8
NEG = -3.0e38


def _gating_body(x_ref, w_ref, b_ref, out_ref, xm_ref):
    bidx = pl.program_id(0)
    xb = x_ref[0].astype(jnp.bfloat16).astype(jnp.float32)
    xm_ref[pl.ds(bidx, 1), :] = jnp.sum(xb, axis=1)[None, :]

    @pl.when(bidx == B - 1)
    def _finish():
        xm = xm_ref[...] * (1.0 / HW)
        pooled = jax.lax.dot_general(
            xm, w_ref[...].astype(jnp.float32), (((1,), (1,)), ((), ())),
            precision=lax.Precision.HIGHEST,
            preferred_element_type=jnp.float32) + b_ref[...]
        iota_e = lax.broadcasted_iota(jnp.int32, (B, E), 1)
        v = pooled
        vals, idxs = [], []
        for _ in range(K):
            m = jnp.max(v, axis=1, keepdims=True)
            im = jnp.min(jnp.where(v == m, iota_e, E), axis=1,
                         keepdims=True)
            vals.append(m)
            idxs.append(im)
            v = jnp.where(iota_e == im, NEG, v)
        exps = [jnp.exp(val - vals[0]) for val in vals]
        denom = functools.reduce(jnp.add, exps)
        out = jnp.zeros((B, E), jnp.float32)
        for ik in range(K):
            out = out + jnp.where(iota_e == idxs[ik], exps[ik] / denom, 0.0)
        out_ref[...] = out


def kernel(x, W, b):
    xr = x.reshape(B, C, HW)
    Wb = W.astype(jnp.bfloat16)
    return pl.pallas_call(
        _gating_body,
        grid=(B,),
        in_specs=[
            pl.BlockSpec((1, C, HW), lambda i: (i, 0, 0)),
            pl.BlockSpec((E, C), lambda i: (0, 0)),
            pl.BlockSpec((1, E), lambda i: (0, 0)),
        ],
        out_specs=pl.BlockSpec((B, E), lambda i: (0, 0)),
        out_shape=jax.ShapeDtypeStruct((B, E), jnp.float32),
        scratch_shapes=[pltpu.VMEM((B, C), jnp.float32)],
        compiler_params=pltpu.CompilerParams(
            dimension_semantics=("arbitrary",)),
    )(xr, Wb, b.reshape(1, E))
